# baseline (speedup 1.0000x reference)
import dataclasses
import functools

import jax
import jax.numpy as jnp
from jax import lax
from jax.experimental import pallas as pl
from jax.experimental.pallas import tpu as pltpu
from jax.experimental.pallas import tpu_sc as plsc

N = 10000
NNZ = 320000
EH = 20000
D_IN = 128
D_HID = 128
H = 64
N_PAD = 10240
EH_PAD = 20480
CHUNK = 128
NCHUNK = NNZ // CHUNK
NW = 32
TRIPS = (NCHUNK + NW - 1) // NW

_f32 = jnp.float32
_i32 = jnp.int32


def _sc_compiler_params():
    cp = pltpu.CompilerParams()
    fields = pltpu.CompilerParams.__dataclass_fields__
    if "needs_layout_passes" in fields:
        cp = dataclasses.replace(cp, needs_layout_passes=False)
    if "use_tc_tiling_on_sc" in fields:
        cp = dataclasses.replace(cp, use_tc_tiling_on_sc=False)
    return cp


def _sc_mesh():
    return plsc.VectorSubcoreMesh(core_axis_name="c", subcore_axis_name="s")


def _mm_body(x_ref, w_ref, o_ref):
    o_ref[...] = jnp.dot(x_ref[...], w_ref[...],
                         preferred_element_type=_f32,
                         precision=lax.Precision.HIGHEST)


def _tc_matmul(X, W):
    B = 1000
    return pl.pallas_call(
        _mm_body,
        grid=(N // B,),
        in_specs=[pl.BlockSpec((B, D_IN), lambda i: (i, 0)),
                  pl.BlockSpec((D_IN, D_HID), lambda i: (0, 0))],
        out_specs=pl.BlockSpec((B, D_HID), lambda i: (i, 0)),
        out_shape=jax.ShapeDtypeStruct((N, D_HID), _f32),
    )(X, W)


def _hist_body(vtx_hbm, edg_hbm, homo_hbm, iota_hbm,
               cnt_out, att_out,
               homo_v, cntloc, attloc, e_idx, v_idx, idrow,
               cnt_sp, att_sp):
    c = lax.axis_index("c")
    s = lax.axis_index("s")
    w = s * 2 + c
    zf = jnp.zeros((16,), _f32)

    @pl.loop(0, EH_PAD // 16)
    def _(r):
        cntloc[r, :] = zf

    @pl.loop(0, N_PAD // 16)
    def _(r):
        attloc[r, :] = zf

    pltpu.sync_copy(cntloc.at[pl.ds(0, 80)], cnt_sp.at[pl.ds(s * 80, 80)])
    pltpu.sync_copy(attloc.at[pl.ds(0, 40)], att_sp.at[pl.ds(s * 40, 40)])

    pltpu.sync_copy(homo_hbm, homo_v)

    lane = lax.iota(_i32, 16)
    ones = jnp.ones((16,), _f32)

    @pl.loop(0, TRIPS)
    def _(t):
        ch = w + t * NW

        @pl.when(ch < NCHUNK)
        def _():
            pltpu.sync_copy(edg_hbm.at[pl.ds(ch * CHUNK, CHUNK)], e_idx)
            pltpu.sync_copy(vtx_hbm.at[pl.ds(ch * CHUNK, CHUNK)], v_idx)
            for k in range(CHUNK // 16):
                ev = e_idx[pl.ds(k * 16, 16)]
                vv = v_idx[pl.ds(k * 16, 16)]
                hv = plsc.load_gather(homo_v, [ev])
                er = lax.shift_right_logical(ev, 4)
                ec = lax.bitwise_and(ev, 15)
                vr = lax.shift_right_logical(vv, 4)
                vc = lax.bitwise_and(vv, 15)
                for l in range(16):
                    m = lane == l
                    plsc.addupdate_scatter(cntloc, [er, ec], ones, mask=m)
                    plsc.addupdate_scatter(attloc, [vr, vc], hv, mask=m)

    plsc.subcore_barrier()

    @pl.loop(0, EH_PAD // 16 // CHUNK)
    def _(j):
        pltpu.sync_copy(iota_hbm.at[pl.ds(j * CHUNK, CHUNK)], idrow)
        pltpu.sync_copy(cntloc.at[pl.ds(j * CHUNK, CHUNK)],
                        cnt_sp.at[idrow], add=True)

    @pl.loop(0, N_PAD // 16 // CHUNK)
    def _(j):
        pltpu.sync_copy(iota_hbm.at[pl.ds(j * CHUNK, CHUNK)], idrow)
        pltpu.sync_copy(attloc.at[pl.ds(j * CHUNK, CHUNK)],
                        att_sp.at[idrow], add=True)

    plsc.subcore_barrier()

    pltpu.sync_copy(cnt_sp.at[pl.ds(s * 80, 80)],
                    cnt_out.at[c, pl.ds(s * 80, 80)])
    pltpu.sync_copy(att_sp.at[pl.ds(s * 40, 40)],
                    att_out.at[c, pl.ds(s * 40, 40)])


def _sc_hist(vertex, edges, homo, iota_rows):
    k = pl.kernel(
        _hist_body,
        out_type=(jax.ShapeDtypeStruct((2, EH_PAD // 16, 16), _f32),
                  jax.ShapeDtypeStruct((2, N_PAD // 16, 16), _f32)),
        mesh=_sc_mesh(),
        scratch_types=[
            pltpu.VMEM((EH,), _f32),
            pltpu.VMEM((EH_PAD // 16, 16), _f32),
            pltpu.VMEM((N_PAD // 16, 16), _f32),
            pltpu.VMEM((CHUNK,), _i32),
            pltpu.VMEM((CHUNK,), _i32),
            pltpu.VMEM((CHUNK,), _i32),
            pltpu.VMEM_SHARED((EH_PAD // 16, 16), _f32),
            pltpu.VMEM_SHARED((N_PAD // 16, 16), _f32),
        ],
        compiler_params=_sc_compiler_params(),
    )
    return k(vertex, edges, homo, iota_rows)


def _seg_body(nrows_tab, nrows_acc, acc_pad, gather_by_vertex,
              tab_hbm, vtx_hbm, edg_hbm,
              out_hbm,
              rows_v, g_idx, s_idx,
              acc_sp):
    c = lax.axis_index("c")
    s = lax.axis_index("s")
    zf = jnp.zeros((16,), _f32)

    @pl.loop(0, CHUNK)
    def _(r):
        for kk in range(H // 16):
            rows_v[r, pl.ds(kk * 16, 16)] = zf

    @pl.loop(0, acc_pad // CHUNK // 16)
    def _(j):
        pltpu.sync_copy(
            rows_v, acc_sp.at[pl.ds((s * (acc_pad // CHUNK // 16) + j) * CHUNK,
                                    CHUNK)])
    plsc.subcore_barrier()

    @pl.when(s == 0)
    def _():
        @pl.loop(0, NCHUNK)
        def _(ch):
            if gather_by_vertex:
                pltpu.sync_copy(vtx_hbm.at[pl.ds(ch * CHUNK, CHUNK)], g_idx)
                pltpu.sync_copy(edg_hbm.at[pl.ds(ch * CHUNK, CHUNK)], s_idx)
            else:
                pltpu.sync_copy(edg_hbm.at[pl.ds(ch * CHUNK, CHUNK)], g_idx)
                pltpu.sync_copy(vtx_hbm.at[pl.ds(ch * CHUNK, CHUNK)], s_idx)
            off = c * nrows_tab
            for k in range(CHUNK // 16):
                g_idx[pl.ds(k * 16, 16)] = g_idx[pl.ds(k * 16, 16)] + off
            pltpu.sync_copy(tab_hbm.at[g_idx], rows_v)
            pltpu.sync_copy(rows_v, acc_sp.at[s_idx], add=True)

    plsc.subcore_barrier()

    nper = acc_pad // 16
    @pl.loop(0, nper // CHUNK)
    def _(j):
        r0 = s * nper + j * CHUNK
        pltpu.sync_copy(acc_sp.at[pl.ds(r0, CHUNK)],
                        out_hbm.at[pl.ds(c * acc_pad + r0, CHUNK)])


def _sc_segsum(tab, vertex, edges, nrows_tab, acc_pad, gather_by_vertex):
    body = functools.partial(_seg_body, nrows_tab, acc_pad, acc_pad,
                             gather_by_vertex)
    k = pl.kernel(
        body,
        out_type=jax.ShapeDtypeStruct((2 * acc_pad, H), _f32),
        mesh=_sc_mesh(),
        scratch_types=[
            pltpu.VMEM((CHUNK, H), _f32),
            pltpu.VMEM((CHUNK,), _i32),
            pltpu.VMEM((CHUNK,), _i32),
            pltpu.VMEM_SHARED((acc_pad, H), _f32),
        ],
        compiler_params=_sc_compiler_params(),
    )
    return k(tab, vertex, edges)


def _scale_body(seg_ref, cnt_ref, homo_ref, o_ref):
    cnt = (cnt_ref[0] + cnt_ref[1]).reshape(-1, 1)
    coef = homo_ref[...] / jnp.maximum(cnt, 1.0)
    o_ref[...] = seg_ref[...] * coef


def _tc_scale(seg, cnt_part, homo_pad):
    B = 2048
    nb = EH_PAD // B
    return pl.pallas_call(
        _scale_body,
        grid=(2, nb),
        in_specs=[pl.BlockSpec((B, H), lambda h, i: (h * nb + i, 0)),
                  pl.BlockSpec((2, B), lambda h, i: (0, i)),
                  pl.BlockSpec((B, 1), lambda h, i: (i, 0))],
        out_specs=pl.BlockSpec((B, H), lambda h, i: (h * nb + i, 0)),
        out_shape=jax.ShapeDtypeStruct((2 * EH_PAD, H), _f32),
    )(seg, cnt_part, homo_pad)


def _comb_body(xp_ref, sa_ref, sb_ref, ap_ref, o_ref):
    att = ap_ref[0] + ap_ref[1]
    Sfull = jnp.concatenate([sa_ref[...], sb_ref[...]], axis=1)
    Xv = jnp.where(att > 0.0, Sfull / jnp.where(att > 0.0, att, 1.0), 0.0)
    o = xp_ref[...] + Xv
    n2 = jnp.sum(o * o, axis=1, keepdims=True)
    o_ref[...] = o * jnp.where(n2 > 0.0, lax.rsqrt(jnp.where(n2 > 0.0, n2, 1.0)), 0.0)


def _tc_combine(Xp, SA, SB, ap):
    B = 2000
    return pl.pallas_call(
        _comb_body,
        grid=(N // B,),
        in_specs=[pl.BlockSpec((B, D_HID), lambda i: (i, 0)),
                  pl.BlockSpec((B, H), lambda i: (i, 0)),
                  pl.BlockSpec((B, H), lambda i: (i, 0)),
                  pl.BlockSpec((2, B, 1), lambda i: (0, i, 0))],
        out_specs=pl.BlockSpec((B, D_HID), lambda i: (i, 0)),
        out_shape=jax.ShapeDtypeStruct((N, D_HID), _f32),
    )(Xp, SA, SB, ap)


def kernel(X, vertex, edges, homo, W):
    vertex = vertex.astype(_i32)
    edges = edges.astype(_i32)
    Xp = _tc_matmul(X, W)
    Xcat = jnp.concatenate([Xp[:, :H], Xp[:, H:]], axis=0)

    iota_rows = jnp.arange(EH_PAD // 16, dtype=_i32)
    cnt_part, att_part = _sc_hist(vertex, edges, homo, iota_rows)

    seg = _sc_segsum(Xcat, vertex, edges, N, EH_PAD, True)

    homo_pad = jnp.pad(homo, (0, EH_PAD - EH)).reshape(EH_PAD, 1)
    Ze = _tc_scale(seg, cnt_part.reshape(2, EH_PAD), homo_pad)

    S = _sc_segsum(Ze, vertex, edges, EH_PAD, N_PAD, False)

    ap = att_part.reshape(2, N_PAD)[:, :N].reshape(2, N, 1)
    out = _tc_combine(Xp, S[:N], S[N_PAD:N_PAD + N], ap)
    return out


# speedup vs baseline: 2.5768x; 2.5768x over previous
import dataclasses
import functools

import jax
import jax.numpy as jnp
from jax import lax
from jax.experimental import pallas as pl
from jax.experimental.pallas import tpu as pltpu
from jax.experimental.pallas import tpu_sc as plsc

N = 10000
NNZ = 320000
EH = 20000
D_IN = 128
D_HID = 128
H = 64
N_PAD = 10240
EH_PAD = 20480
CHUNK = 128
NCHUNK = NNZ // CHUNK
NW = 32
TRIPS = (NCHUNK + NW - 1) // NW

_f32 = jnp.float32
_i32 = jnp.int32


def _sc_compiler_params():
    cp = pltpu.CompilerParams()
    fields = pltpu.CompilerParams.__dataclass_fields__
    if "needs_layout_passes" in fields:
        cp = dataclasses.replace(cp, needs_layout_passes=False)
    if "use_tc_tiling_on_sc" in fields:
        cp = dataclasses.replace(cp, use_tc_tiling_on_sc=False)
    return cp


def _sc_mesh():
    return plsc.VectorSubcoreMesh(core_axis_name="c", subcore_axis_name="s")


def _mm_body(x_ref, w_ref, o_ref):
    o_ref[...] = jnp.dot(x_ref[...], w_ref[...],
                         preferred_element_type=_f32,
                         precision=lax.Precision.HIGHEST)


def _tc_matmul(X, W):
    B = 1000
    return pl.pallas_call(
        _mm_body,
        grid=(N // B,),
        in_specs=[pl.BlockSpec((B, D_IN), lambda i: (i, 0)),
                  pl.BlockSpec((D_IN, D_HID), lambda i: (0, 0))],
        out_specs=pl.BlockSpec((B, D_HID), lambda i: (i, 0)),
        out_shape=jax.ShapeDtypeStruct((N, D_HID), _f32),
    )(X, W)


def _hist_body(vtx_hbm, edg_hbm, homo_hbm, iota_hbm,
               cnt_out, att_out,
               homo_v, cntloc, attloc, e_idx, v_idx, idrow,
               cnt_sp, att_sp):
    c = lax.axis_index("c")
    s = lax.axis_index("s")
    w = s * 2 + c
    zf = jnp.zeros((16,), _f32)

    @pl.loop(0, EH_PAD // 16)
    def _(r):
        cntloc[r, :] = zf

    @pl.loop(0, N_PAD // 16)
    def _(r):
        attloc[r, :] = zf

    pltpu.sync_copy(cntloc.at[pl.ds(0, 80)], cnt_sp.at[pl.ds(s * 80, 80)])
    pltpu.sync_copy(attloc.at[pl.ds(0, 40)], att_sp.at[pl.ds(s * 40, 40)])

    pltpu.sync_copy(homo_hbm, homo_v)

    lane = lax.iota(_i32, 16)
    ones = jnp.ones((16,), _f32)

    @pl.loop(0, TRIPS)
    def _(t):
        ch = w + t * NW

        @pl.when(ch < NCHUNK)
        def _():
            pltpu.sync_copy(edg_hbm.at[pl.ds(ch * CHUNK, CHUNK)], e_idx)
            pltpu.sync_copy(vtx_hbm.at[pl.ds(ch * CHUNK, CHUNK)], v_idx)
            for k in range(CHUNK // 16):
                ev = e_idx[pl.ds(k * 16, 16)]
                vv = v_idx[pl.ds(k * 16, 16)]
                hv = plsc.load_gather(homo_v, [ev])
                er = lax.shift_right_logical(ev, 4)
                ec = lax.bitwise_and(ev, 15)
                vr = lax.shift_right_logical(vv, 4)
                vc = lax.bitwise_and(vv, 15)
                for l in range(16):
                    m = lane == l
                    plsc.addupdate_scatter(cntloc, [er, ec], ones, mask=m)
                    plsc.addupdate_scatter(attloc, [vr, vc], hv, mask=m)

    plsc.subcore_barrier()

    @pl.loop(0, EH_PAD // 16 // CHUNK)
    def _(j):
        pltpu.sync_copy(iota_hbm.at[pl.ds(j * CHUNK, CHUNK)], idrow)
        pltpu.sync_copy(cntloc.at[pl.ds(j * CHUNK, CHUNK)],
                        cnt_sp.at[idrow], add=True)

    @pl.loop(0, N_PAD // 16 // CHUNK)
    def _(j):
        pltpu.sync_copy(iota_hbm.at[pl.ds(j * CHUNK, CHUNK)], idrow)
        pltpu.sync_copy(attloc.at[pl.ds(j * CHUNK, CHUNK)],
                        att_sp.at[idrow], add=True)

    plsc.subcore_barrier()

    pltpu.sync_copy(cnt_sp.at[pl.ds(s * 80, 80)],
                    cnt_out.at[c, pl.ds(s * 80, 80)])
    pltpu.sync_copy(att_sp.at[pl.ds(s * 40, 40)],
                    att_out.at[c, pl.ds(s * 40, 40)])


def _sc_hist(vertex, edges, homo, iota_rows):
    k = pl.kernel(
        _hist_body,
        out_type=(jax.ShapeDtypeStruct((2, EH_PAD // 16, 16), _f32),
                  jax.ShapeDtypeStruct((2, N_PAD // 16, 16), _f32)),
        mesh=_sc_mesh(),
        scratch_types=[
            pltpu.VMEM((EH,), _f32),
            pltpu.VMEM((EH_PAD // 16, 16), _f32),
            pltpu.VMEM((N_PAD // 16, 16), _f32),
            pltpu.VMEM((CHUNK,), _i32),
            pltpu.VMEM((CHUNK,), _i32),
            pltpu.VMEM((CHUNK,), _i32),
            pltpu.VMEM_SHARED((EH_PAD // 16, 16), _f32),
            pltpu.VMEM_SHARED((N_PAD // 16, 16), _f32),
        ],
        compiler_params=_sc_compiler_params(),
    )
    return k(vertex, edges, homo, iota_rows)


B_CH = 10
NBLK = NCHUNK // B_CH


def _seg_body(nrows_tab, acc_pad, gather_by_vertex,
              tab_hbm, vtx_hbm, edg_hbm,
              out_hbm,
              rows_a, rows_b, g_buf, s_buf,
              sem_a, sem_b,
              acc_sp):
    c = lax.axis_index("c")
    s = lax.axis_index("s")
    zf = jnp.zeros((16,), _f32)

    @pl.loop(0, CHUNK)
    def _(r):
        for kk in range(H // 16):
            rows_a[r, pl.ds(kk * 16, 16)] = zf

    @pl.loop(0, acc_pad // CHUNK // 16)
    def _(j):
        pltpu.sync_copy(
            rows_a, acc_sp.at[pl.ds((s * (acc_pad // CHUNK // 16) + j) * CHUNK,
                                    CHUNK)])
    plsc.subcore_barrier()

    @pl.when(s == 0)
    def _():
        @pl.loop(0, NBLK)
        def _(b):
            if gather_by_vertex:
                pltpu.sync_copy(vtx_hbm.at[pl.ds(b * B_CH, B_CH)], g_buf)
                pltpu.sync_copy(edg_hbm.at[pl.ds(b * B_CH, B_CH)], s_buf)
            else:
                pltpu.sync_copy(edg_hbm.at[pl.ds(b * B_CH, B_CH)], g_buf)
                pltpu.sync_copy(vtx_hbm.at[pl.ds(b * B_CH, B_CH)], s_buf)
            off = c * nrows_tab
            for j in range(B_CH):
                for k in range(CHUNK // 16):
                    g_buf[j, pl.ds(k * 16, 16)] = (
                        g_buf[j, pl.ds(k * 16, 16)] + off)
            bufs = (rows_a, rows_b)
            sems = (sem_a, sem_b)
            pend = pltpu.async_copy(tab_hbm.at[g_buf.at[0]], bufs[0], sems[0])
            for j in range(B_CH):
                cur = bufs[j % 2]
                if j + 1 < B_CH:
                    nxt = pltpu.async_copy(tab_hbm.at[g_buf.at[j + 1]],
                                           bufs[(j + 1) % 2],
                                           sems[(j + 1) % 2])
                pend.wait()
                pltpu.sync_copy(cur, acc_sp.at[s_buf.at[j]], add=True)
                if j + 1 < B_CH:
                    pend = nxt

    plsc.subcore_barrier()

    nper = acc_pad // 16
    @pl.loop(0, nper // CHUNK)
    def _(j):
        r0 = s * nper + j * CHUNK
        pltpu.sync_copy(acc_sp.at[pl.ds(r0, CHUNK)],
                        out_hbm.at[pl.ds(c * acc_pad + r0, CHUNK)])


def _sc_segsum(tab, vtx2, edg2, nrows_tab, acc_pad, gather_by_vertex):
    body = functools.partial(_seg_body, nrows_tab, acc_pad, gather_by_vertex)
    k = pl.kernel(
        body,
        out_type=jax.ShapeDtypeStruct((2 * acc_pad, H), _f32),
        mesh=_sc_mesh(),
        scratch_types=[
            pltpu.VMEM((CHUNK, H), _f32),
            pltpu.VMEM((CHUNK, H), _f32),
            pltpu.VMEM((B_CH, CHUNK), _i32),
            pltpu.VMEM((B_CH, CHUNK), _i32),
            pltpu.SemaphoreType.DMA,
            pltpu.SemaphoreType.DMA,
            pltpu.VMEM_SHARED((acc_pad, H), _f32),
        ],
        compiler_params=_sc_compiler_params(),
    )
    return k(tab, vtx2, edg2)


def _scale_body(seg_ref, cnt_ref, homo_ref, o_ref):
    cnt = (cnt_ref[0] + cnt_ref[1]).reshape(-1, 1)
    coef = homo_ref[...] / jnp.maximum(cnt, 1.0)
    o_ref[...] = seg_ref[...] * coef


def _tc_scale(seg, cnt_part, homo_pad):
    B = 2048
    nb = EH_PAD // B
    return pl.pallas_call(
        _scale_body,
        grid=(2, nb),
        in_specs=[pl.BlockSpec((B, H), lambda h, i: (h * nb + i, 0)),
                  pl.BlockSpec((2, B), lambda h, i: (0, i)),
                  pl.BlockSpec((B, 1), lambda h, i: (i, 0))],
        out_specs=pl.BlockSpec((B, H), lambda h, i: (h * nb + i, 0)),
        out_shape=jax.ShapeDtypeStruct((2 * EH_PAD, H), _f32),
    )(seg, cnt_part, homo_pad)


def _comb_body(xp_ref, sa_ref, sb_ref, ap_ref, o_ref):
    att = ap_ref[0] + ap_ref[1]
    Sfull = jnp.concatenate([sa_ref[...], sb_ref[...]], axis=1)
    Xv = jnp.where(att > 0.0, Sfull / jnp.where(att > 0.0, att, 1.0), 0.0)
    o = xp_ref[...] + Xv
    n2 = jnp.sum(o * o, axis=1, keepdims=True)
    o_ref[...] = o * jnp.where(n2 > 0.0, lax.rsqrt(jnp.where(n2 > 0.0, n2, 1.0)), 0.0)


def _tc_combine(Xp, SA, SB, ap):
    B = 2000
    return pl.pallas_call(
        _comb_body,
        grid=(N // B,),
        in_specs=[pl.BlockSpec((B, D_HID), lambda i: (i, 0)),
                  pl.BlockSpec((B, H), lambda i: (i, 0)),
                  pl.BlockSpec((B, H), lambda i: (i, 0)),
                  pl.BlockSpec((2, B, 1), lambda i: (0, i, 0))],
        out_specs=pl.BlockSpec((B, D_HID), lambda i: (i, 0)),
        out_shape=jax.ShapeDtypeStruct((N, D_HID), _f32),
    )(Xp, SA, SB, ap)


def kernel(X, vertex, edges, homo, W):
    vertex = vertex.astype(_i32)
    edges = edges.astype(_i32)
    Xp = _tc_matmul(X, W)
    Xcat = jnp.concatenate([Xp[:, :H], Xp[:, H:]], axis=0)

    iota_rows = jnp.arange(EH_PAD // 16, dtype=_i32)
    cnt_part, att_part = _sc_hist(vertex, edges, homo, iota_rows)

    vtx2 = vertex.reshape(NCHUNK, CHUNK)
    edg2 = edges.reshape(NCHUNK, CHUNK)
    seg = _sc_segsum(Xcat, vtx2, edg2, N, EH_PAD, True)

    homo_pad = jnp.pad(homo, (0, EH_PAD - EH)).reshape(EH_PAD, 1)
    Ze = _tc_scale(seg, cnt_part.reshape(2, EH_PAD), homo_pad)

    S = _sc_segsum(Ze, vtx2, edg2, EH_PAD, N_PAD, False)

    ap = att_part.reshape(2, N_PAD)[:, :N].reshape(2, N, 1)
    out = _tc_combine(Xp, S[:N], S[N_PAD:N_PAD + N], ap)
    return out


# speedup vs baseline: 3.1947x; 1.2398x over previous
import dataclasses
import functools

import jax
import jax.numpy as jnp
from jax import lax
from jax.experimental import pallas as pl
from jax.experimental.pallas import tpu as pltpu
from jax.experimental.pallas import tpu_sc as plsc

N = 10000
NNZ = 320000
EH = 20000
D_IN = 128
D_HID = 128
H = 64
N_PAD = 10240
EH_PAD = 20480
CHUNK = 128
NCHUNK = NNZ // CHUNK
NW = 32
TRIPS = (NCHUNK + NW - 1) // NW

_f32 = jnp.float32
_i32 = jnp.int32


def _sc_compiler_params():
    cp = pltpu.CompilerParams()
    fields = pltpu.CompilerParams.__dataclass_fields__
    if "needs_layout_passes" in fields:
        cp = dataclasses.replace(cp, needs_layout_passes=False)
    if "use_tc_tiling_on_sc" in fields:
        cp = dataclasses.replace(cp, use_tc_tiling_on_sc=False)
    return cp


def _sc_mesh():
    return plsc.VectorSubcoreMesh(core_axis_name="c", subcore_axis_name="s")


def _mm_body(x_ref, w_ref, o_ref):
    o_ref[...] = jnp.dot(x_ref[...], w_ref[...],
                         preferred_element_type=_f32,
                         precision=lax.Precision.HIGHEST)


def _tc_matmul(X, W):
    B = 1000
    return pl.pallas_call(
        _mm_body,
        grid=(N // B,),
        in_specs=[pl.BlockSpec((B, D_IN), lambda i: (i, 0)),
                  pl.BlockSpec((D_IN, D_HID), lambda i: (0, 0))],
        out_specs=pl.BlockSpec((B, D_HID), lambda i: (i, 0)),
        out_shape=jax.ShapeDtypeStruct((N, D_HID), _f32),
    )(X, W)


def _hist_body(vtx_hbm, edg_hbm, homo_hbm, iota_hbm,
               cnt_out, att_out,
               homo_v, cntloc, attloc, e_idx, v_idx, idrow,
               cnt_sp, att_sp):
    c = lax.axis_index("c")
    s = lax.axis_index("s")
    w = s * 2 + c
    zf = jnp.zeros((16,), _f32)

    @pl.loop(0, EH_PAD // 16)
    def _(r):
        cntloc[r, :] = zf

    @pl.loop(0, N_PAD // 16)
    def _(r):
        attloc[r, :] = zf

    pltpu.sync_copy(cntloc.at[pl.ds(0, 80)], cnt_sp.at[pl.ds(s * 80, 80)])
    pltpu.sync_copy(attloc.at[pl.ds(0, 40)], att_sp.at[pl.ds(s * 40, 40)])

    pltpu.sync_copy(homo_hbm, homo_v)

    lane = lax.iota(_i32, 16)
    ones = jnp.ones((16,), _f32)

    @pl.loop(0, TRIPS)
    def _(t):
        ch = w + t * NW

        @pl.when(ch < NCHUNK)
        def _():
            pltpu.sync_copy(edg_hbm.at[pl.ds(ch * CHUNK, CHUNK)], e_idx)
            pltpu.sync_copy(vtx_hbm.at[pl.ds(ch * CHUNK, CHUNK)], v_idx)
            for k in range(CHUNK // 16):
                ev = e_idx[pl.ds(k * 16, 16)]
                vv = v_idx[pl.ds(k * 16, 16)]
                hv = plsc.load_gather(homo_v, [ev])
                er = lax.shift_right_logical(ev, 4)
                ec = lax.bitwise_and(ev, 15)
                vr = lax.shift_right_logical(vv, 4)
                vc = lax.bitwise_and(vv, 15)
                for l in range(16):
                    m = lane == l
                    plsc.addupdate_scatter(cntloc, [er, ec], ones, mask=m)
                    plsc.addupdate_scatter(attloc, [vr, vc], hv, mask=m)

    plsc.subcore_barrier()

    @pl.loop(0, EH_PAD // 16 // CHUNK)
    def _(j):
        pltpu.sync_copy(iota_hbm.at[pl.ds(j * CHUNK, CHUNK)], idrow)
        pltpu.sync_copy(cntloc.at[pl.ds(j * CHUNK, CHUNK)],
                        cnt_sp.at[idrow], add=True)

    @pl.loop(0, N_PAD // 16 // CHUNK)
    def _(j):
        pltpu.sync_copy(iota_hbm.at[pl.ds(j * CHUNK, CHUNK)], idrow)
        pltpu.sync_copy(attloc.at[pl.ds(j * CHUNK, CHUNK)],
                        att_sp.at[idrow], add=True)

    plsc.subcore_barrier()

    pltpu.sync_copy(cnt_sp.at[pl.ds(s * 80, 80)],
                    cnt_out.at[c, pl.ds(s * 80, 80)])
    pltpu.sync_copy(att_sp.at[pl.ds(s * 40, 40)],
                    att_out.at[c, pl.ds(s * 40, 40)])


def _sc_hist(vertex, edges, homo, iota_rows):
    k = pl.kernel(
        _hist_body,
        out_type=(jax.ShapeDtypeStruct((2, EH_PAD // 16, 16), _f32),
                  jax.ShapeDtypeStruct((2, N_PAD // 16, 16), _f32)),
        mesh=_sc_mesh(),
        scratch_types=[
            pltpu.VMEM((EH,), _f32),
            pltpu.VMEM((EH_PAD // 16, 16), _f32),
            pltpu.VMEM((N_PAD // 16, 16), _f32),
            pltpu.VMEM((CHUNK,), _i32),
            pltpu.VMEM((CHUNK,), _i32),
            pltpu.VMEM((CHUNK,), _i32),
            pltpu.VMEM_SHARED((EH_PAD // 16, 16), _f32),
            pltpu.VMEM_SHARED((N_PAD // 16, 16), _f32),
        ],
        compiler_params=_sc_compiler_params(),
    )
    return k(vertex, edges, homo, iota_rows)


B_CH = 10
NBLK = NCHUNK // B_CH


def _seg_body(nrows_tab, acc_pad, gather_by_vertex,
              tab_hbm, vtx_hbm, edg_hbm,
              out_hbm,
              rows_a, rows_b, rows_c, rows_d, g_buf, s_buf,
              sem_a, sem_b, sem_c, sem_d,
              acc_sp):
    c = lax.axis_index("c")
    s = lax.axis_index("s")
    zf = jnp.zeros((16,), _f32)

    @pl.loop(0, CHUNK)
    def _(r):
        for kk in range(H // 16):
            rows_a[r, pl.ds(kk * 16, 16)] = zf

    @pl.loop(0, acc_pad // CHUNK // 16)
    def _(j):
        pltpu.sync_copy(
            rows_a, acc_sp.at[pl.ds((s * (acc_pad // CHUNK // 16) + j) * CHUNK,
                                    CHUNK)])
    plsc.subcore_barrier()

    @pl.when(s == 0)
    def _():
        @pl.loop(0, NBLK)
        def _(b):
            if gather_by_vertex:
                pltpu.sync_copy(vtx_hbm.at[pl.ds(b * B_CH, B_CH)], g_buf)
                pltpu.sync_copy(edg_hbm.at[pl.ds(b * B_CH, B_CH)], s_buf)
            else:
                pltpu.sync_copy(edg_hbm.at[pl.ds(b * B_CH, B_CH)], g_buf)
                pltpu.sync_copy(vtx_hbm.at[pl.ds(b * B_CH, B_CH)], s_buf)
            off = c * nrows_tab
            for j in range(B_CH):
                for k in range(CHUNK // 16):
                    g_buf[j, pl.ds(k * 16, 16)] = (
                        g_buf[j, pl.ds(k * 16, 16)] + off)
            bufs = (rows_a, rows_b, rows_c, rows_d)
            sems = (sem_a, sem_b, sem_c, sem_d)
            depth = 4
            pend = [pltpu.async_copy(tab_hbm.at[g_buf.at[j]], bufs[j], sems[j])
                    for j in range(depth)]
            for j in range(B_CH):
                pend[j % depth].wait()
                pltpu.sync_copy(bufs[j % depth], acc_sp.at[s_buf.at[j]],
                                add=True)
                if j + depth < B_CH:
                    pend[j % depth] = pltpu.async_copy(
                        tab_hbm.at[g_buf.at[j + depth]], bufs[j % depth],
                        sems[j % depth])

    plsc.subcore_barrier()

    nper = acc_pad // 16
    @pl.loop(0, nper // CHUNK)
    def _(j):
        r0 = s * nper + j * CHUNK
        pltpu.sync_copy(acc_sp.at[pl.ds(r0, CHUNK)],
                        out_hbm.at[pl.ds(c * acc_pad + r0, CHUNK)])


def _sc_segsum(tab, vtx2, edg2, nrows_tab, acc_pad, gather_by_vertex):
    body = functools.partial(_seg_body, nrows_tab, acc_pad, gather_by_vertex)
    k = pl.kernel(
        body,
        out_type=jax.ShapeDtypeStruct((2 * acc_pad, H), _f32),
        mesh=_sc_mesh(),
        scratch_types=[
            pltpu.VMEM((CHUNK, H), _f32),
            pltpu.VMEM((CHUNK, H), _f32),
            pltpu.VMEM((CHUNK, H), _f32),
            pltpu.VMEM((CHUNK, H), _f32),
            pltpu.VMEM((B_CH, CHUNK), _i32),
            pltpu.VMEM((B_CH, CHUNK), _i32),
            pltpu.SemaphoreType.DMA,
            pltpu.SemaphoreType.DMA,
            pltpu.SemaphoreType.DMA,
            pltpu.SemaphoreType.DMA,
            pltpu.VMEM_SHARED((acc_pad, H), _f32),
        ],
        compiler_params=_sc_compiler_params(),
    )
    return k(tab, vtx2, edg2)


def _scale_body(seg_ref, cnt_ref, homo_ref, o_ref):
    cnt = (cnt_ref[0] + cnt_ref[1]).reshape(-1, 1)
    coef = homo_ref[...] / jnp.maximum(cnt, 1.0)
    o_ref[...] = seg_ref[...] * coef


def _tc_scale(seg, cnt_part, homo_pad):
    B = 2048
    nb = EH_PAD // B
    return pl.pallas_call(
        _scale_body,
        grid=(2, nb),
        in_specs=[pl.BlockSpec((B, H), lambda h, i: (h * nb + i, 0)),
                  pl.BlockSpec((2, B), lambda h, i: (0, i)),
                  pl.BlockSpec((B, 1), lambda h, i: (i, 0))],
        out_specs=pl.BlockSpec((B, H), lambda h, i: (h * nb + i, 0)),
        out_shape=jax.ShapeDtypeStruct((2 * EH_PAD, H), _f32),
    )(seg, cnt_part, homo_pad)


def _comb_body(xp_ref, sa_ref, sb_ref, ap_ref, o_ref):
    att = ap_ref[0] + ap_ref[1]
    Sfull = jnp.concatenate([sa_ref[...], sb_ref[...]], axis=1)
    Xv = jnp.where(att > 0.0, Sfull / jnp.where(att > 0.0, att, 1.0), 0.0)
    o = xp_ref[...] + Xv
    n2 = jnp.sum(o * o, axis=1, keepdims=True)
    o_ref[...] = o * jnp.where(n2 > 0.0, lax.rsqrt(jnp.where(n2 > 0.0, n2, 1.0)), 0.0)


def _tc_combine(Xp, SA, SB, ap):
    B = 2000
    return pl.pallas_call(
        _comb_body,
        grid=(N // B,),
        in_specs=[pl.BlockSpec((B, D_HID), lambda i: (i, 0)),
                  pl.BlockSpec((B, H), lambda i: (i, 0)),
                  pl.BlockSpec((B, H), lambda i: (i, 0)),
                  pl.BlockSpec((2, B, 1), lambda i: (0, i, 0))],
        out_specs=pl.BlockSpec((B, D_HID), lambda i: (i, 0)),
        out_shape=jax.ShapeDtypeStruct((N, D_HID), _f32),
    )(Xp, SA, SB, ap)


def kernel(X, vertex, edges, homo, W):
    vertex = vertex.astype(_i32)
    edges = edges.astype(_i32)
    Xp = _tc_matmul(X, W)
    Xcat = jnp.concatenate([Xp[:, :H], Xp[:, H:]], axis=0)

    iota_rows = jnp.arange(EH_PAD // 16, dtype=_i32)
    cnt_part, att_part = _sc_hist(vertex, edges, homo, iota_rows)

    vtx2 = vertex.reshape(NCHUNK, CHUNK)
    edg2 = edges.reshape(NCHUNK, CHUNK)
    seg = _sc_segsum(Xcat, vtx2, edg2, N, EH_PAD, True)

    homo_pad = jnp.pad(homo, (0, EH_PAD - EH)).reshape(EH_PAD, 1)
    Ze = _tc_scale(seg, cnt_part.reshape(2, EH_PAD), homo_pad)

    S = _sc_segsum(Ze, vtx2, edg2, EH_PAD, N_PAD, False)

    ap = att_part.reshape(2, N_PAD)[:, :N].reshape(2, N, 1)
    out = _tc_combine(Xp, S[:N], S[N_PAD:N_PAD + N], ap)
    return out


# speedup vs baseline: 3.9519x; 1.2370x over previous
import dataclasses
import functools

import jax
import jax.numpy as jnp
from jax import lax
from jax.experimental import pallas as pl
from jax.experimental.pallas import tpu as pltpu
from jax.experimental.pallas import tpu_sc as plsc

N = 10000
NNZ = 320000
EH = 20000
D_IN = 128
D_HID = 128
H = 64
N_PAD = 10240
EH_PAD = 20480
CHUNK = 128
NCHUNK = NNZ // CHUNK
NW = 32
TRIPS = (NCHUNK + NW - 1) // NW

_f32 = jnp.float32
_i32 = jnp.int32


def _sc_compiler_params():
    cp = pltpu.CompilerParams()
    fields = pltpu.CompilerParams.__dataclass_fields__
    if "needs_layout_passes" in fields:
        cp = dataclasses.replace(cp, needs_layout_passes=False)
    if "use_tc_tiling_on_sc" in fields:
        cp = dataclasses.replace(cp, use_tc_tiling_on_sc=False)
    return cp


def _sc_mesh():
    return plsc.VectorSubcoreMesh(core_axis_name="c", subcore_axis_name="s")


def _mm_body(x_ref, w_ref, o_ref):
    o_ref[...] = jnp.dot(x_ref[...], w_ref[...],
                         preferred_element_type=_f32,
                         precision=lax.Precision.HIGHEST)


def _tc_matmul(X, W):
    B = 1000
    return pl.pallas_call(
        _mm_body,
        grid=(N // B,),
        in_specs=[pl.BlockSpec((B, D_IN), lambda i: (i, 0)),
                  pl.BlockSpec((D_IN, D_HID), lambda i: (0, 0))],
        out_specs=pl.BlockSpec((B, D_HID), lambda i: (i, 0)),
        out_shape=jax.ShapeDtypeStruct((N, D_HID), _f32),
    )(X, W)


def _hist_body(vtx_hbm, edg_hbm, homo_hbm, iota_hbm,
               cnt_out, att_out,
               homo_v, cntloc, attloc, e_idx, v_idx, idrow,
               cnt_sp, att_sp):
    c = lax.axis_index("c")
    s = lax.axis_index("s")
    w = s * 2 + c
    zf = jnp.zeros((16,), _f32)

    @pl.loop(0, EH_PAD // 16)
    def _(r):
        cntloc[r, :] = zf

    @pl.loop(0, N_PAD // 16)
    def _(r):
        attloc[r, :] = zf

    pltpu.sync_copy(cntloc.at[pl.ds(0, 80)], cnt_sp.at[pl.ds(s * 80, 80)])
    pltpu.sync_copy(attloc.at[pl.ds(0, 40)], att_sp.at[pl.ds(s * 40, 40)])

    pltpu.sync_copy(homo_hbm, homo_v)

    lane = lax.iota(_i32, 16)
    ones = jnp.ones((16,), _f32)

    @pl.loop(0, TRIPS)
    def _(t):
        ch = w + t * NW

        @pl.when(ch < NCHUNK)
        def _():
            pltpu.sync_copy(edg_hbm.at[pl.ds(ch * CHUNK, CHUNK)], e_idx)
            pltpu.sync_copy(vtx_hbm.at[pl.ds(ch * CHUNK, CHUNK)], v_idx)
            for k in range(CHUNK // 16):
                ev = e_idx[pl.ds(k * 16, 16)]
                vv = v_idx[pl.ds(k * 16, 16)]
                hv = plsc.load_gather(homo_v, [ev])
                er = lax.shift_right_logical(ev, 4)
                ec = lax.bitwise_and(ev, 15)
                vr = lax.shift_right_logical(vv, 4)
                vc = lax.bitwise_and(vv, 15)
                for l in range(16):
                    m = lane == l
                    plsc.addupdate_scatter(cntloc, [er, ec], ones, mask=m)
                    plsc.addupdate_scatter(attloc, [vr, vc], hv, mask=m)

    plsc.subcore_barrier()

    @pl.loop(0, EH_PAD // 16 // CHUNK)
    def _(j):
        pltpu.sync_copy(iota_hbm.at[pl.ds(j * CHUNK, CHUNK)], idrow)
        pltpu.sync_copy(cntloc.at[pl.ds(j * CHUNK, CHUNK)],
                        cnt_sp.at[idrow], add=True)

    @pl.loop(0, N_PAD // 16 // CHUNK)
    def _(j):
        pltpu.sync_copy(iota_hbm.at[pl.ds(j * CHUNK, CHUNK)], idrow)
        pltpu.sync_copy(attloc.at[pl.ds(j * CHUNK, CHUNK)],
                        att_sp.at[idrow], add=True)

    plsc.subcore_barrier()

    pltpu.sync_copy(cnt_sp.at[pl.ds(s * 80, 80)],
                    cnt_out.at[c, pl.ds(s * 80, 80)])
    pltpu.sync_copy(att_sp.at[pl.ds(s * 40, 40)],
                    att_out.at[c, pl.ds(s * 40, 40)])


def _sc_hist(vertex, edges, homo, iota_rows):
    k = pl.kernel(
        _hist_body,
        out_type=(jax.ShapeDtypeStruct((2, EH_PAD // 16, 16), _f32),
                  jax.ShapeDtypeStruct((2, N_PAD // 16, 16), _f32)),
        mesh=_sc_mesh(),
        scratch_types=[
            pltpu.VMEM((EH,), _f32),
            pltpu.VMEM((EH_PAD // 16, 16), _f32),
            pltpu.VMEM((N_PAD // 16, 16), _f32),
            pltpu.VMEM((CHUNK,), _i32),
            pltpu.VMEM((CHUNK,), _i32),
            pltpu.VMEM((CHUNK,), _i32),
            pltpu.VMEM_SHARED((EH_PAD // 16, 16), _f32),
            pltpu.VMEM_SHARED((N_PAD // 16, 16), _f32),
        ],
        compiler_params=_sc_compiler_params(),
    )
    return k(vertex, edges, homo, iota_rows)


B_CH = 20
NBLK = NCHUNK // B_CH


def _seg_body(nrows_tab, acc_pad, gather_by_vertex,
              tab_hbm, vtx_hbm, edg_hbm,
              out_hbm,
              rows_a, rows_b, rows_c, rows_d, g_buf, s_buf,
              sem_a, sem_b, sem_c, sem_d,
              acc_sp):
    c = lax.axis_index("c")
    s = lax.axis_index("s")
    zf = jnp.zeros((16,), _f32)

    @pl.loop(0, CHUNK)
    def _(r):
        for kk in range(H // 16):
            rows_a[r, pl.ds(kk * 16, 16)] = zf

    @pl.loop(0, acc_pad // CHUNK // 16)
    def _(j):
        pltpu.sync_copy(
            rows_a, acc_sp.at[pl.ds((s * (acc_pad // CHUNK // 16) + j) * CHUNK,
                                    CHUNK)])
    plsc.subcore_barrier()

    @pl.when(s == 0)
    def _():
        @pl.loop(0, NBLK)
        def _(b):
            if gather_by_vertex:
                pltpu.sync_copy(vtx_hbm.at[pl.ds(b * B_CH, B_CH)], g_buf)
                pltpu.sync_copy(edg_hbm.at[pl.ds(b * B_CH, B_CH)], s_buf)
            else:
                pltpu.sync_copy(edg_hbm.at[pl.ds(b * B_CH, B_CH)], g_buf)
                pltpu.sync_copy(vtx_hbm.at[pl.ds(b * B_CH, B_CH)], s_buf)
            off = c * nrows_tab
            for j in range(B_CH):
                for k in range(CHUNK // 16):
                    g_buf[j, pl.ds(k * 16, 16)] = (
                        g_buf[j, pl.ds(k * 16, 16)] + off)
            bufs = (rows_a, rows_b, rows_c, rows_d)
            sems = (sem_a, sem_b, sem_c, sem_d)
            depth = 4
            pend = [pltpu.async_copy(tab_hbm.at[g_buf.at[j]], bufs[j], sems[j])
                    for j in range(depth)]
            for j in range(B_CH):
                pend[j % depth].wait()
                pltpu.sync_copy(bufs[j % depth], acc_sp.at[s_buf.at[j]],
                                add=True)
                if j + depth < B_CH:
                    pend[j % depth] = pltpu.async_copy(
                        tab_hbm.at[g_buf.at[j + depth]], bufs[j % depth],
                        sems[j % depth])

    plsc.subcore_barrier()

    nper = acc_pad // 16
    @pl.loop(0, nper // CHUNK)
    def _(j):
        r0 = s * nper + j * CHUNK
        pltpu.sync_copy(acc_sp.at[pl.ds(r0, CHUNK)],
                        out_hbm.at[pl.ds(c * acc_pad + r0, CHUNK)])


def _sc_segsum(tab, vtx2, edg2, nrows_tab, acc_pad, gather_by_vertex):
    body = functools.partial(_seg_body, nrows_tab, acc_pad, gather_by_vertex)
    k = pl.kernel(
        body,
        out_type=jax.ShapeDtypeStruct((2 * acc_pad, H), _f32),
        mesh=_sc_mesh(),
        scratch_types=[
            pltpu.VMEM((CHUNK, H), _f32),
            pltpu.VMEM((CHUNK, H), _f32),
            pltpu.VMEM((CHUNK, H), _f32),
            pltpu.VMEM((CHUNK, H), _f32),
            pltpu.VMEM((B_CH, CHUNK), _i32),
            pltpu.VMEM((B_CH, CHUNK), _i32),
            pltpu.SemaphoreType.DMA,
            pltpu.SemaphoreType.DMA,
            pltpu.SemaphoreType.DMA,
            pltpu.SemaphoreType.DMA,
            pltpu.VMEM_SHARED((acc_pad, H), _f32),
        ],
        compiler_params=_sc_compiler_params(),
    )
    return k(tab, vtx2, edg2)


def _scale_body(seg_ref, cnt_ref, homo_ref, o_ref):
    cnt = (cnt_ref[0] + cnt_ref[1]).reshape(-1, 1)
    coef = homo_ref[...] / jnp.maximum(cnt, 1.0)
    o_ref[...] = seg_ref[...] * coef


def _tc_scale(seg, cnt_part, homo_pad):
    B = 2048
    nb = EH_PAD // B
    return pl.pallas_call(
        _scale_body,
        grid=(2, nb),
        in_specs=[pl.BlockSpec((B, H), lambda h, i: (h * nb + i, 0)),
                  pl.BlockSpec((2, B), lambda h, i: (0, i)),
                  pl.BlockSpec((B, 1), lambda h, i: (i, 0))],
        out_specs=pl.BlockSpec((B, H), lambda h, i: (h * nb + i, 0)),
        out_shape=jax.ShapeDtypeStruct((2 * EH_PAD, H), _f32),
    )(seg, cnt_part, homo_pad)


def _comb_body(xp_ref, sa_ref, sb_ref, ap_ref, o_ref):
    att = ap_ref[0] + ap_ref[1]
    Sfull = jnp.concatenate([sa_ref[...], sb_ref[...]], axis=1)
    Xv = jnp.where(att > 0.0, Sfull / jnp.where(att > 0.0, att, 1.0), 0.0)
    o = xp_ref[...] + Xv
    n2 = jnp.sum(o * o, axis=1, keepdims=True)
    o_ref[...] = o * jnp.where(n2 > 0.0, lax.rsqrt(jnp.where(n2 > 0.0, n2, 1.0)), 0.0)


def _tc_combine(Xp, SA, SB, ap):
    B = 2000
    return pl.pallas_call(
        _comb_body,
        grid=(N // B,),
        in_specs=[pl.BlockSpec((B, D_HID), lambda i: (i, 0)),
                  pl.BlockSpec((B, H), lambda i: (i, 0)),
                  pl.BlockSpec((B, H), lambda i: (i, 0)),
                  pl.BlockSpec((2, B, 1), lambda i: (0, i, 0))],
        out_specs=pl.BlockSpec((B, D_HID), lambda i: (i, 0)),
        out_shape=jax.ShapeDtypeStruct((N, D_HID), _f32),
    )(Xp, SA, SB, ap)


def kernel(X, vertex, edges, homo, W):
    vertex = vertex.astype(_i32)
    edges = edges.astype(_i32)
    Xp = _tc_matmul(X, W)
    Xcat = jnp.concatenate([Xp[:, :H], Xp[:, H:]], axis=0)

    iota_rows = jnp.arange(EH_PAD // 16, dtype=_i32)
    cnt_part, att_part = _sc_hist(vertex, edges, homo, iota_rows)

    vtx2 = vertex.reshape(NCHUNK, CHUNK)
    edg2 = edges.reshape(NCHUNK, CHUNK)
    seg = _sc_segsum(Xcat, vtx2, edg2, N, EH_PAD, True)

    homo_pad = jnp.pad(homo, (0, EH_PAD - EH)).reshape(EH_PAD, 1)
    Ze = _tc_scale(seg, cnt_part.reshape(2, EH_PAD), homo_pad)

    S = _sc_segsum(Ze, vtx2, edg2, EH_PAD, N_PAD, False)

    ap = att_part.reshape(2, N_PAD)[:, :N].reshape(2, N, 1)
    out = _tc_combine(Xp, S[:N], S[N_PAD:N_PAD + N], ap)
    return out
